# baseline (speedup 1.0000x reference)
import jax
import jax.numpy as jnp
from jax.experimental import pallas as pl
from jax.experimental.pallas import tpu as pltpu

BM = 1024
BN = 1024


def _edge_kernel(a_ref, b_ref, o_ref):
    i = pl.program_id(0)
    j = pl.program_id(1)
    acc = jax.lax.dot_general(
        a_ref[...], b_ref[...],
        dimension_numbers=(((1,), (1,)), ((), ())),
        preferred_element_type=jnp.float32,
    )

    @pl.when(i == j)
    def _mask_diag():
        row = jax.lax.broadcasted_iota(jnp.int32, (BM, BN), 0)
        col = jax.lax.broadcasted_iota(jnp.int32, (BM, BN), 1)
        o_ref[...] = jnp.where(row == col, 0.0, acc)

    @pl.when(i != j)
    def _plain():
        o_ref[...] = acc


def kernel(h):
    n, d = h.shape
    grid = (n // BM, n // BN)
    return pl.pallas_call(
        _edge_kernel,
        grid=grid,
        in_specs=[
            pl.BlockSpec((BM, d), lambda i, j: (i, 0)),
            pl.BlockSpec((BN, d), lambda i, j: (j, 0)),
        ],
        out_specs=pl.BlockSpec((BM, BN), lambda i, j: (i, j)),
        out_shape=jax.ShapeDtypeStruct((n, n), jnp.float32),
        compiler_params=pltpu.CompilerParams(
            dimension_semantics=("parallel", "arbitrary"),
        ),
    )(h, h)


# speedup vs baseline: 1.3235x; 1.3235x over previous
import jax
import jax.numpy as jnp
from jax.experimental import pallas as pl
from jax.experimental.pallas import tpu as pltpu

BM = 512


def _edge_kernel(a_ref, b_ref, o_ref):
    i = pl.program_id(0)
    n = b_ref.shape[0]
    acc = jax.lax.dot_general(
        a_ref[...], b_ref[...],
        dimension_numbers=(((1,), (1,)), ((), ())),
        preferred_element_type=jnp.float32,
    )
    row = jax.lax.broadcasted_iota(jnp.int32, (BM, n), 0) + i * BM
    col = jax.lax.broadcasted_iota(jnp.int32, (BM, n), 1)
    o_ref[...] = jnp.where(row == col, 0.0, acc)


def kernel(h):
    n, d = h.shape
    grid = (n // BM,)
    return pl.pallas_call(
        _edge_kernel,
        grid=grid,
        in_specs=[
            pl.BlockSpec((BM, d), lambda i: (i, 0)),
            pl.BlockSpec((n, d), lambda i: (0, 0)),
        ],
        out_specs=pl.BlockSpec((BM, n), lambda i: (i, 0)),
        out_shape=jax.ShapeDtypeStruct((n, n), jnp.float32),
        compiler_params=pltpu.CompilerParams(
            dimension_semantics=("parallel",),
        ),
    )(h, h)


# speedup vs baseline: 1.3369x; 1.0101x over previous
import jax
import jax.numpy as jnp
from jax.experimental import pallas as pl
from jax.experimental.pallas import tpu as pltpu

BM = 256


def _edge_kernel(a_ref, b_ref, o_ref):
    i = pl.program_id(0)
    n = b_ref.shape[0]
    acc = jax.lax.dot_general(
        a_ref[...], b_ref[...],
        dimension_numbers=(((1,), (1,)), ((), ())),
        preferred_element_type=jnp.float32,
    )
    row = jax.lax.broadcasted_iota(jnp.int32, (BM, n), 0) + i * BM
    col = jax.lax.broadcasted_iota(jnp.int32, (BM, n), 1)
    o_ref[...] = jnp.where(row == col, 0.0, acc)


def kernel(h):
    n, d = h.shape
    grid = (n // BM,)
    return pl.pallas_call(
        _edge_kernel,
        grid=grid,
        in_specs=[
            pl.BlockSpec((BM, d), lambda i: (i, 0)),
            pl.BlockSpec((n, d), lambda i: (0, 0)),
        ],
        out_specs=pl.BlockSpec((BM, n), lambda i: (i, 0)),
        out_shape=jax.ShapeDtypeStruct((n, n), jnp.float32),
        compiler_params=pltpu.CompilerParams(
            dimension_semantics=("parallel",),
        ),
    )(h, h)
